# baseline (speedup 1.0000x reference)
import functools

import jax
import jax.numpy as jnp
from jax import lax
from jax.experimental import pallas as pl
from jax.experimental.pallas import tpu as pltpu
from jax.experimental.pallas import tpu_sc as plsc

_F = 26
_FDIM = 100000
_B = 16384
_NW = 32
_BPW = _B // _NW
_IPW = _BPW * _F


def _sc_kernel(x_hbm, t_hbm, bias_hbm, out_hbm, idx_v, val_v, bias_v, o_v,
               semx, semg):
    info = plsc.get_sparse_core_info()
    nc = info.num_cores
    wid = lax.axis_index("s") * nc + lax.axis_index("c")
    base = wid * _BPW

    xcopies = [
        pltpu.async_copy(
            x_hbm.at[pl.ds(f * _B + base, _BPW)],
            idx_v.at[pl.ds(f * _BPW, _BPW)],
            semx,
        )
        for f in range(_F)
    ]
    pltpu.sync_copy(bias_hbm, bias_v)
    biasv = bias_v[...]

    def seed(cb, carry):
        o_v[pl.ds(cb * 16, 16)] = biasv
        return carry

    lax.fori_loop(0, _BPW // 16, seed, 0)

    gathers = []
    for f in range(_F):
        xcopies[f].wait()
        gathers.append(pltpu.async_copy(
            t_hbm.at[pl.ds(f * _FDIM, _FDIM)].at[
                idx_v.at[pl.ds(f * _BPW, _BPW)]],
            val_v.at[pl.ds(f * _BPW, _BPW)],
            semg,
        ))

    for f in range(_F):
        gathers[f].wait()

        def acc_field(cb, carry, f=f):
            b0 = cb * 16
            o_v[pl.ds(b0, 16)] = (
                o_v[pl.ds(b0, 16)] + val_v[pl.ds(f * _BPW + b0, 16)])
            return carry

        lax.fori_loop(0, _BPW // 16, acc_field, 0)

    pltpu.sync_copy(o_v, out_hbm.at[pl.ds(base, _BPW)])


def kernel(x, table, bias):
    xt_flat = x.astype(jnp.int32).T.reshape(-1)
    tflat = table.reshape(-1)
    bias16 = jnp.broadcast_to(bias.astype(jnp.float32), (16,))

    mesh = plsc.VectorSubcoreMesh(core_axis_name="c", subcore_axis_name="s")
    run = functools.partial(
        pl.kernel,
        out_type=jax.ShapeDtypeStruct((_B,), jnp.float32),
        mesh=mesh,
        scratch_types=[
            pltpu.VMEM((_IPW,), jnp.int32),
            pltpu.VMEM((_IPW,), jnp.float32),
            pltpu.VMEM((16,), jnp.float32),
            pltpu.VMEM((_BPW,), jnp.float32),
            pltpu.SemaphoreType.DMA,
            pltpu.SemaphoreType.DMA,
        ],
    )(_sc_kernel)
    out = run(xt_flat, tflat, bias16)
    return out.reshape(_B, 1)


# speedup vs baseline: 1.0213x; 1.0213x over previous
import functools

import jax
import jax.numpy as jnp
from jax import lax
from jax.experimental import pallas as pl
from jax.experimental.pallas import tpu as pltpu
from jax.experimental.pallas import tpu_sc as plsc

_F = 26
_FDIM = 100000
_B = 16384
_NW = 32
_BPW = _B // _NW
_IPW = _BPW * _F


def _sc_kernel(x_hbm, t_hbm, bias_hbm, out_hbm, idx_v, val_v, bias_v, o_v,
               semx, semg):
    info = plsc.get_sparse_core_info()
    nc = info.num_cores
    wid = lax.axis_index("s") * nc + lax.axis_index("c")
    base = wid * _BPW

    xcopies = [
        pltpu.async_copy(
            x_hbm.at[pl.ds(f * _B + base, _BPW)],
            idx_v.at[pl.ds(f * _BPW, _BPW)],
            semx,
        )
        for f in range(_F)
    ]
    pltpu.sync_copy(bias_hbm, bias_v)
    biasv = bias_v[...]

    def seed(cb, carry):
        o_v[pl.ds(cb * 16, 16)] = biasv
        return carry

    lax.fori_loop(0, _BPW // 16, seed, 0)

    for c in xcopies:
        c.wait()

    gathers = [
        pltpu.async_copy(
            t_hbm.at[pl.ds(f * _FDIM, _FDIM)].at[
                idx_v.at[pl.ds(f * _BPW, _BPW)]],
            val_v.at[pl.ds(f * _BPW, _BPW)],
            semg,
        )
        for f in range(_F)
    ]
    for g in gathers:
        g.wait()

    def acc(cb, carry):
        b0 = cb * 16
        a = o_v[pl.ds(b0, 16)]
        for f in range(_F):
            a = a + val_v[pl.ds(f * _BPW + b0, 16)]
        o_v[pl.ds(b0, 16)] = a
        return carry

    lax.fori_loop(0, _BPW // 16, acc, 0)

    pltpu.sync_copy(o_v, out_hbm.at[pl.ds(base, _BPW)])


def kernel(x, table, bias):
    xt_flat = x.astype(jnp.int32).T.reshape(-1)
    tflat = table.reshape(-1)
    bias16 = jnp.broadcast_to(bias.astype(jnp.float32), (16,))

    mesh = plsc.VectorSubcoreMesh(core_axis_name="c", subcore_axis_name="s")
    run = functools.partial(
        pl.kernel,
        out_type=jax.ShapeDtypeStruct((_B,), jnp.float32),
        mesh=mesh,
        scratch_types=[
            pltpu.VMEM((_IPW,), jnp.int32),
            pltpu.VMEM((_IPW,), jnp.float32),
            pltpu.VMEM((16,), jnp.float32),
            pltpu.VMEM((_BPW,), jnp.float32),
            pltpu.SemaphoreType.DMA,
            pltpu.SemaphoreType.DMA,
        ],
    )(_sc_kernel)
    out = run(xt_flat, tflat, bias16)
    return out.reshape(_B, 1)


# speedup vs baseline: 1.0213x; 1.0000x over previous
import functools

import jax
import jax.numpy as jnp
from jax import lax
from jax.experimental import pallas as pl
from jax.experimental.pallas import tpu as pltpu
from jax.experimental.pallas import tpu_sc as plsc

_F = 26
_FDIM = 100000
_B = 16384
_NW = 32
_BPW = _B // _NW
_IPW = _BPW * _F


def _sc_kernel(x_hbm, t_hbm, bias_hbm, out_hbm, idx_v, val_v, bias_v, o_v,
               semx, semg):
    info = plsc.get_sparse_core_info()
    nc = info.num_cores
    wid = lax.axis_index("s") * nc + lax.axis_index("c")
    base = wid * _BPW

    xcopies = [
        pltpu.async_copy(
            x_hbm.at[pl.ds(f * _B + base, _BPW)],
            idx_v.at[pl.ds(f * _BPW, _BPW)],
            semx,
        )
        for f in range(_F)
    ]
    pltpu.sync_copy(bias_hbm, bias_v)
    biasv = bias_v[...]

    for c in xcopies:
        c.wait()

    gathers = [
        pltpu.async_copy(
            t_hbm.at[pl.ds(f * _FDIM, _FDIM)].at[
                idx_v.at[pl.ds(f * _BPW, _BPW)]],
            val_v.at[pl.ds(f * _BPW, _BPW)],
            semg,
        )
        for f in range(_F)
    ]
    for g in gathers:
        g.wait()

    def acc(cb, carry):
        b0 = cb * 16
        a = biasv
        for f in range(_F):
            a = a + val_v[pl.ds(f * _BPW + b0, 16)]
        o_v[pl.ds(b0, 16)] = a
        return carry

    lax.fori_loop(0, _BPW // 16, acc, 0)

    pltpu.sync_copy(o_v, out_hbm.at[pl.ds(base, _BPW)])


def kernel(x, table, bias):
    xt_flat = x.astype(jnp.int32).T.reshape(-1)
    tflat = table.reshape(-1)
    bias16 = jnp.broadcast_to(bias.astype(jnp.float32), (16,))

    mesh = plsc.VectorSubcoreMesh(core_axis_name="c", subcore_axis_name="s")
    run = functools.partial(
        pl.kernel,
        out_type=jax.ShapeDtypeStruct((_B,), jnp.float32),
        mesh=mesh,
        scratch_types=[
            pltpu.VMEM((_IPW,), jnp.int32),
            pltpu.VMEM((_IPW,), jnp.float32),
            pltpu.VMEM((16,), jnp.float32),
            pltpu.VMEM((_BPW,), jnp.float32),
            pltpu.SemaphoreType.DMA,
            pltpu.SemaphoreType.DMA,
        ],
    )(_sc_kernel)
    out = run(xt_flat, tflat, bias16)
    return out.reshape(_B, 1)
